# Initial kernel scaffold; baseline (speedup 1.0000x reference)
import jax
import jax.numpy as jnp
from jax.experimental import pallas as pl


def kernel(v_s, c_s, v_sem, c_sem, v_class, c_class, Wqkv_v, bqkv_v, Wo_v, bo_v, Wqkv_c, bqkv_c, Wo_c, bo_c):
    raise NotImplementedError("write your pallas kernel here")


# speedup vs baseline: 3.8483x; 3.8483x over previous
import functools

import jax
import jax.numpy as jnp
from jax import lax
from jax.experimental import pallas as pl
from jax.experimental.pallas import tpu as pltpu
from jax.experimental.pallas import tpu_sc as plsc

EMB = 128
NCLS = 16
NHEADS = 4
HD = EMB // NHEADS
BLK = 128
CNTW = 128

NCORES = 2
NSUB = 16
NW = NCORES * NSUB


@functools.cache
def _mesh():
    return plsc.VectorSubcoreMesh(core_axis_name="core", subcore_axis_name="subcore")


def _fill2d(ref, rows, cols, value):
    val = jnp.full((1, 16), value, jnp.float32)

    @pl.loop(0, rows)
    def _(i):
        for j in range(cols // 16):
            ref.at[pl.ds(i, 1), pl.ds(j * 16, 16)][...] = val


def _segment_sums_sc(v_s, c_s, v_class2, c_class2):
    nv = v_s.shape[0]
    nc = c_s.shape[0]
    nfull_v, tail_v = nv // BLK, nv % BLK
    nfull_c, tail_c = nc // BLK, nc % BLK
    assert tail_v % 8 == 0 and tail_c % 8 == 0

    out_type = [
        jax.ShapeDtypeStruct((NCORES, NCLS, EMB), jnp.float32),
        jax.ShapeDtypeStruct((NCORES, NCLS, CNTW), jnp.float32),
        jax.ShapeDtypeStruct((NCORES, NCLS, EMB), jnp.float32),
        jax.ShapeDtypeStruct((NCORES, NCLS, CNTW), jnp.float32),
    ]
    scratch_types = [
        pltpu.VMEM_SHARED((NCLS, EMB), jnp.float32),
        pltpu.VMEM_SHARED((NCLS, CNTW), jnp.float32),
        pltpu.VMEM_SHARED((NCLS, EMB), jnp.float32),
        pltpu.VMEM_SHARED((NCLS, CNTW), jnp.float32),
        pltpu.VMEM((BLK, EMB), jnp.float32),
        pltpu.VMEM((BLK,), jnp.int32),
        pltpu.VMEM((BLK, CNTW), jnp.float32),
        pltpu.VMEM((NCLS, EMB), jnp.float32),
        pltpu.VMEM((NCLS, CNTW), jnp.float32),
        pltpu.VMEM((tail_v,), jnp.int32) if tail_v else None,
        pltpu.VMEM((tail_v, EMB), jnp.float32) if tail_v else None,
        pltpu.VMEM((tail_c,), jnp.int32) if tail_c else None,
        pltpu.VMEM((tail_c, EMB), jnp.float32) if tail_c else None,
    ]
    scratch_types = [s for s in scratch_types if s is not None]

    @functools.partial(pl.kernel, mesh=_mesh(), out_type=out_type,
                       scratch_types=scratch_types)
    def kern(v_s_hbm, c_s_hbm, vcls_hbm, ccls_hbm,
             vsum_hbm, vcnt_hbm, csum_hbm, ccnt_hbm,
             vacc_sh, vcntacc_sh, cacc_sh, ccntacc_sh,
             rows_v, idx_v, ones_v, zero_wide, zero_narrow,
             *tails):
        cid = lax.axis_index("core")
        sid = lax.axis_index("subcore")
        wid = sid * NCORES + cid

        _fill2d(ones_v, BLK, CNTW, 1.0)

        @pl.when(sid == 0)
        def _():
            _fill2d(zero_wide, NCLS, EMB, 0.0)
            _fill2d(zero_narrow, NCLS, CNTW, 0.0)
            pltpu.sync_copy(zero_wide, vacc_sh)
            pltpu.sync_copy(zero_narrow, vcntacc_sh)
            pltpu.sync_copy(zero_wide, cacc_sh)
            pltpu.sync_copy(zero_narrow, ccntacc_sh)

        plsc.subcore_barrier()

        @pl.loop(wid, nfull_v, step=NW)
        def _(blk):
            base = blk * BLK
            pltpu.sync_copy(vcls_hbm.at[0, pl.ds(base, BLK)], idx_v)
            pltpu.sync_copy(v_s_hbm.at[pl.ds(base, BLK)], rows_v)
            pltpu.sync_copy(rows_v, vacc_sh.at[idx_v], add=True)
            pltpu.sync_copy(ones_v, vcntacc_sh.at[idx_v], add=True)

        @pl.loop(wid, nfull_c, step=NW)
        def _(blk):
            base = blk * BLK
            pltpu.sync_copy(ccls_hbm.at[0, pl.ds(base, BLK)], idx_v)
            pltpu.sync_copy(c_s_hbm.at[pl.ds(base, BLK)], rows_v)
            pltpu.sync_copy(rows_v, cacc_sh.at[idx_v], add=True)
            pltpu.sync_copy(ones_v, ccntacc_sh.at[idx_v], add=True)

        ti = 0
        if tail_v:
            vtidx, vtrows = tails[ti], tails[ti + 1]
            ti += 2

            @pl.when((cid == 0) & (sid == 1))
            def _():
                pltpu.sync_copy(vcls_hbm.at[0, pl.ds(nfull_v * BLK, tail_v)], vtidx)
                pltpu.sync_copy(v_s_hbm.at[pl.ds(nfull_v * BLK, tail_v)], vtrows)
                pltpu.sync_copy(vtrows, vacc_sh.at[vtidx], add=True)
                pltpu.sync_copy(ones_v.at[pl.ds(0, tail_v)], vcntacc_sh.at[vtidx],
                                add=True)
        if tail_c:
            ctidx, ctrows = tails[ti], tails[ti + 1]

            @pl.when((cid == 1) & (sid == 1))
            def _():
                pltpu.sync_copy(ccls_hbm.at[0, pl.ds(nfull_c * BLK, tail_c)], ctidx)
                pltpu.sync_copy(c_s_hbm.at[pl.ds(nfull_c * BLK, tail_c)], ctrows)
                pltpu.sync_copy(ctrows, cacc_sh.at[ctidx], add=True)
                pltpu.sync_copy(ones_v.at[pl.ds(0, tail_c)], ccntacc_sh.at[ctidx],
                                add=True)

        plsc.subcore_barrier()

        @pl.when(sid == 0)
        def _():
            pltpu.sync_copy(vacc_sh, vsum_hbm.at[cid])
            pltpu.sync_copy(vcntacc_sh, vcnt_hbm.at[cid])
            pltpu.sync_copy(cacc_sh, csum_hbm.at[cid])
            pltpu.sync_copy(ccntacc_sh, ccnt_hbm.at[cid])

    return kern(v_s, c_s, v_class2, c_class2)


def _mha_body(sem, sum2, cnt2, Wqkv, bqkv, Wo, bo):
    fea_sum = sum2[0] + sum2[1]
    cnt = cnt2[0][:, 0:1] + cnt2[1][:, 0:1] + 1e-8
    fea = fea_sum / cnt

    dn_t = (((1,), (1,)), ((), ()))
    q = lax.dot_general(sem, Wqkv[0:EMB], dn_t,
                        preferred_element_type=jnp.float32) + bqkv[0, 0:EMB]
    k = lax.dot_general(fea, Wqkv[EMB:2 * EMB], dn_t,
                        preferred_element_type=jnp.float32) + bqkv[0, EMB:2 * EMB]
    v = lax.dot_general(fea, Wqkv[2 * EMB:3 * EMB], dn_t,
                        preferred_element_type=jnp.float32) + bqkv[0, 2 * EMB:3 * EMB]

    outs = []
    scale = 1.0 / (HD ** 0.5)
    for h in range(NHEADS):
        qh = q[:, h * HD:(h + 1) * HD]
        kh = k[:, h * HD:(h + 1) * HD]
        vh = v[:, h * HD:(h + 1) * HD]
        scores = lax.dot_general(qh, kh, dn_t,
                                 preferred_element_type=jnp.float32) * scale
        m = jnp.max(scores, axis=-1, keepdims=True)
        e = jnp.exp(scores - m)
        attn = e / jnp.sum(e, axis=-1, keepdims=True)
        outs.append(lax.dot_general(attn, vh, (((1,), (0,)), ((), ())),
                                    preferred_element_type=jnp.float32))
    o = jnp.concatenate(outs, axis=1)
    return lax.dot_general(o, Wo, dn_t,
                           preferred_element_type=jnp.float32) + bo[0]


def _attn_tc(vsum, vcnt, csum, ccnt, v_sem, c_sem,
             Wqkv_v, bqkv_v, Wo_v, bo_v, Wqkv_c, bqkv_c, Wo_c, bo_c):
    def body(vsum_r, vcnt_r, csum_r, ccnt_r, vsem_r, csem_r,
             wqv_r, bqv_r, wov_r, bov_r, wqc_r, bqc_r, woc_r, boc_r,
             vfin_r, cfin_r):
        vfin_r[...] = _mha_body(vsem_r[...], vsum_r[...], vcnt_r[...],
                                wqv_r[...], bqv_r[...], wov_r[...], bov_r[...])
        cfin_r[...] = _mha_body(csem_r[...], csum_r[...], ccnt_r[...],
                                wqc_r[...], bqc_r[...], woc_r[...], boc_r[...])

    return pl.pallas_call(
        body,
        out_shape=[jax.ShapeDtypeStruct((NCLS, EMB), jnp.float32),
                   jax.ShapeDtypeStruct((NCLS, EMB), jnp.float32)],
    )(vsum, vcnt, csum, ccnt, v_sem, c_sem,
      Wqkv_v, bqkv_v.reshape(1, -1), Wo_v, bo_v.reshape(1, -1),
      Wqkv_c, bqkv_c.reshape(1, -1), Wo_c, bo_c.reshape(1, -1))


def _gather_sc(vfin, cfin, v_class2, c_class2, nv, nc):
    nfull_v, tail_v = nv // BLK, nv % BLK
    nfull_c, tail_c = nc // BLK, nc % BLK

    out_type = [
        jax.ShapeDtypeStruct((nv, EMB), jnp.float32),
        jax.ShapeDtypeStruct((nc, EMB), jnp.float32),
    ]
    scratch_types = [
        pltpu.VMEM_SHARED((NCLS, EMB), jnp.float32),
        pltpu.VMEM_SHARED((NCLS, EMB), jnp.float32),
        pltpu.VMEM((BLK, EMB), jnp.float32),
        pltpu.VMEM((BLK,), jnp.int32),
        pltpu.VMEM((tail_v,), jnp.int32) if tail_v else None,
        pltpu.VMEM((tail_v, EMB), jnp.float32) if tail_v else None,
        pltpu.VMEM((tail_c,), jnp.int32) if tail_c else None,
        pltpu.VMEM((tail_c, EMB), jnp.float32) if tail_c else None,
    ]
    scratch_types = [s for s in scratch_types if s is not None]

    @functools.partial(pl.kernel, mesh=_mesh(), out_type=out_type,
                       scratch_types=scratch_types)
    def kern(vfin_hbm, cfin_hbm, vcls_hbm, ccls_hbm, vout_hbm, cout_hbm,
             vfin_sh, cfin_sh, rows_v, idx_v, *tails):
        cid = lax.axis_index("core")
        sid = lax.axis_index("subcore")
        wid = sid * NCORES + cid

        @pl.when(sid == 0)
        def _():
            pltpu.sync_copy(vfin_hbm, vfin_sh)
            pltpu.sync_copy(cfin_hbm, cfin_sh)

        plsc.subcore_barrier()

        @pl.loop(wid, nfull_v, step=NW)
        def _(blk):
            base = blk * BLK
            pltpu.sync_copy(vcls_hbm.at[0, pl.ds(base, BLK)], idx_v)
            pltpu.sync_copy(vfin_sh.at[idx_v], rows_v)
            pltpu.sync_copy(rows_v, vout_hbm.at[pl.ds(base, BLK)])

        @pl.loop(wid, nfull_c, step=NW)
        def _(blk):
            base = blk * BLK
            pltpu.sync_copy(ccls_hbm.at[0, pl.ds(base, BLK)], idx_v)
            pltpu.sync_copy(cfin_sh.at[idx_v], rows_v)
            pltpu.sync_copy(rows_v, cout_hbm.at[pl.ds(base, BLK)])

        ti = 0
        if tail_v:
            vtidx, vtrows = tails[ti], tails[ti + 1]
            ti += 2

            @pl.when((cid == 0) & (sid == 1))
            def _():
                pltpu.sync_copy(vcls_hbm.at[0, pl.ds(nfull_v * BLK, tail_v)], vtidx)
                pltpu.sync_copy(vfin_sh.at[vtidx], vtrows)
                pltpu.sync_copy(vtrows, vout_hbm.at[pl.ds(nfull_v * BLK, tail_v)])
        if tail_c:
            ctidx, ctrows = tails[ti], tails[ti + 1]

            @pl.when((cid == 1) & (sid == 1))
            def _():
                pltpu.sync_copy(ccls_hbm.at[0, pl.ds(nfull_c * BLK, tail_c)], ctidx)
                pltpu.sync_copy(cfin_sh.at[ctidx], ctrows)
                pltpu.sync_copy(ctrows, cout_hbm.at[pl.ds(nfull_c * BLK, tail_c)])

    return kern(vfin, cfin, v_class2, c_class2)


def kernel(v_s, c_s, v_sem, c_sem, v_class, c_class,
           Wqkv_v, bqkv_v, Wo_v, bo_v, Wqkv_c, bqkv_c, Wo_c, bo_c):
    nv = v_s.shape[0]
    nc = c_s.shape[0]
    v_class2 = v_class.reshape(1, nv)
    c_class2 = c_class.reshape(1, nc)

    vsum, vcnt, csum, ccnt = _segment_sums_sc(v_s, c_s, v_class2, c_class2)
    vfin, cfin = _attn_tc(vsum, vcnt, csum, ccnt, v_sem, c_sem,
                          Wqkv_v, bqkv_v, Wo_v, bo_v,
                          Wqkv_c, bqkv_c, Wo_c, bo_c)
    v_updates, c_updates = _gather_sc(vfin, cfin, v_class2, c_class2, nv, nc)
    return (v_updates, c_updates)


# speedup vs baseline: 6.0251x; 1.5656x over previous
import functools

import jax
import jax.numpy as jnp
from jax import lax
from jax.experimental import pallas as pl
from jax.experimental.pallas import tpu as pltpu
from jax.experimental.pallas import tpu_sc as plsc

EMB = 128
NCLS = 16
NHEADS = 4
HD = EMB // NHEADS
BLK = 128
KSUB = 2
CH = KSUB * BLK
CNTW = 128

NCORES = 2
NSUB = 16
NW = NCORES * NSUB


@functools.cache
def _mesh():
    return plsc.VectorSubcoreMesh(core_axis_name="core", subcore_axis_name="subcore")


def _splits(n):
    nch = n // CH
    rem = n - nch * CH
    nx = rem // BLK
    tail = rem - nx * BLK
    assert tail % 8 == 0
    return nch, nx, tail


def _fill2d(ref, rows, cols, value):
    val = jnp.full((1, 16), value, jnp.float32)

    @pl.loop(0, rows)
    def _(i):
        for j in range(cols // 16):
            ref.at[pl.ds(i, 1), pl.ds(j * 16, 16)][...] = val


def _segment_sums_sc(v_s, c_s, vcls2d, ccls2d, vcls1d, ccls1d):
    nv = v_s.shape[0]
    nc = c_s.shape[0]
    nch_v, nx_v, tail_v = _splits(nv)
    nch_c, nx_c, tail_c = _splits(nc)

    out_type = [
        jax.ShapeDtypeStruct((NCORES, NCLS, EMB), jnp.float32),
        jax.ShapeDtypeStruct((NCORES, NCLS, CNTW), jnp.float32),
        jax.ShapeDtypeStruct((NCORES, NCLS, EMB), jnp.float32),
        jax.ShapeDtypeStruct((NCORES, NCLS, CNTW), jnp.float32),
    ]
    scratch_types = [
        pltpu.VMEM_SHARED((NCLS, EMB), jnp.float32),
        pltpu.VMEM_SHARED((NCLS, CNTW), jnp.float32),
        pltpu.VMEM_SHARED((NCLS, EMB), jnp.float32),
        pltpu.VMEM_SHARED((NCLS, CNTW), jnp.float32),
        pltpu.VMEM((CH, EMB), jnp.float32),
        pltpu.VMEM((CH, EMB), jnp.float32),
        pltpu.VMEM((KSUB, BLK), jnp.int32),
        pltpu.VMEM((KSUB, BLK), jnp.int32),
        pltpu.VMEM((BLK, CNTW), jnp.float32),
        pltpu.VMEM((NCLS, EMB), jnp.float32),
        pltpu.SemaphoreType.DMA,
        pltpu.SemaphoreType.DMA,
        pltpu.SemaphoreType.DMA,
        pltpu.SemaphoreType.DMA,
        pltpu.SemaphoreType.DMA,
        pltpu.SemaphoreType.DMA,
    ]
    if nx_v:
        scratch_types += [pltpu.VMEM((KSUB, BLK), jnp.int32),
                          pltpu.VMEM((BLK, EMB), jnp.float32)]
    if tail_v:
        scratch_types += [pltpu.VMEM((tail_v,), jnp.int32),
                          pltpu.VMEM((tail_v, EMB), jnp.float32)]
    if tail_c:
        scratch_types += [pltpu.VMEM((tail_c,), jnp.int32),
                          pltpu.VMEM((tail_c, EMB), jnp.float32)]

    @functools.partial(pl.kernel, mesh=_mesh(), out_type=out_type,
                       scratch_types=scratch_types)
    def kern(v_s_hbm, c_s_hbm, vcls2_hbm, ccls2_hbm, vcls1_hbm, ccls1_hbm,
             vsum_hbm, vcnt_hbm, csum_hbm, ccnt_hbm,
             vacc_sh, vcntacc_sh, cacc_sh, ccntacc_sh,
             rows0, rows1, idx0, idx1, ones_v, zero_w,
             s_i0, s_r0, s_i1, s_r1, s_s0, s_s1, *extra):
        cid = lax.axis_index("core")
        sid = lax.axis_index("subcore")
        wid = sid * NCORES + cid

        _fill2d(ones_v, BLK, CNTW, 1.0)

        @pl.when(sid == 0)
        def _():
            _fill2d(zero_w, NCLS, EMB, 0.0)
            pltpu.sync_copy(zero_w, vacc_sh)
            pltpu.sync_copy(zero_w, vcntacc_sh)
            pltpu.sync_copy(zero_w, cacc_sh)
            pltpu.sync_copy(zero_w, ccntacc_sh)

        plsc.subcore_barrier()

        def branch(rows_hbm, cls2_hbm, acc_sh, cnt_sh, nch):
            def start(ch, idxb, rowsb, s_i, s_r):
                base = pl.multiple_of(ch * CH, CH)
                pltpu.async_copy(cls2_hbm.at[ch], idxb, s_i)
                pltpu.async_copy(rows_hbm.at[pl.ds(base, CH)], rowsb, s_r)

            def wait(ch, idxb, rowsb, s_i, s_r):
                base = pl.multiple_of(ch * CH, CH)
                pltpu.make_async_copy(cls2_hbm.at[ch], idxb, s_i).wait()
                pltpu.make_async_copy(
                    rows_hbm.at[pl.ds(base, CH)], rowsb, s_r).wait()

            def work(ch, idxb, rowsb, s_i, s_r, s_s):
                wait(ch, idxb, rowsb, s_i, s_r)
                hs = []
                for j in range(KSUB):
                    hs.append(pltpu.async_copy(
                        rowsb.at[pl.ds(j * BLK, BLK)],
                        acc_sh.at[idxb.at[j]], s_s, add=True))
                    hs.append(pltpu.async_copy(
                        ones_v, cnt_sh.at[idxb.at[j]], s_s, add=True))
                for h in hs:
                    h.wait()

                @pl.when(ch + 2 * NW < nch)
                def _():
                    start(ch + 2 * NW, idxb, rowsb, s_i, s_r)

            @pl.when(wid < nch)
            def _():
                start(wid, idx0, rows0, s_i0, s_r0)

            @pl.when(wid + NW < nch)
            def _():
                start(wid + NW, idx1, rows1, s_i1, s_r1)

            @pl.loop(wid, nch, step=2 * NW)
            def _(ch):
                work(ch, idx0, rows0, s_i0, s_r0, s_s0)

                @pl.when(ch + NW < nch)
                def _():
                    work(ch + NW, idx1, rows1, s_i1, s_r1, s_s1)

        branch(v_s_hbm, vcls2_hbm, vacc_sh, vcntacc_sh, nch_v)
        branch(c_s_hbm, ccls2_hbm, cacc_sh, ccntacc_sh, nch_c)

        ei = 0
        for x in range(nx_v):
            e_idx, e_rows = extra[ei], extra[ei + 1]
            ei += 2
            base = nch_v * CH + x * BLK

            @pl.when((cid == 0) & (sid == 2 + x))
            def _():
                pltpu.sync_copy(vcls2_hbm.at[nch_v], e_idx)
                pltpu.sync_copy(v_s_hbm.at[pl.ds(base, BLK)], e_rows)
                pltpu.sync_copy(e_rows, vacc_sh.at[e_idx.at[x]], add=True)
                pltpu.sync_copy(ones_v, vcntacc_sh.at[e_idx.at[x]], add=True)

        if tail_v:
            t_idx, t_rows = extra[ei], extra[ei + 1]
            ei += 2
            base = nch_v * CH + nx_v * BLK

            @pl.when((cid == 0) & (sid == 1))
            def _():
                pltpu.sync_copy(vcls1_hbm.at[0, pl.ds(base, tail_v)], t_idx)
                pltpu.sync_copy(v_s_hbm.at[pl.ds(base, tail_v)], t_rows)
                pltpu.sync_copy(t_rows, vacc_sh.at[t_idx], add=True)
                pltpu.sync_copy(ones_v.at[pl.ds(0, tail_v)], vcntacc_sh.at[t_idx],
                                add=True)
        if tail_c:
            t_idx, t_rows = extra[ei], extra[ei + 1]
            base = nch_c * CH + nx_c * BLK

            @pl.when((cid == 1) & (sid == 1))
            def _():
                pltpu.sync_copy(ccls1_hbm.at[0, pl.ds(base, tail_c)], t_idx)
                pltpu.sync_copy(c_s_hbm.at[pl.ds(base, tail_c)], t_rows)
                pltpu.sync_copy(t_rows, cacc_sh.at[t_idx], add=True)
                pltpu.sync_copy(ones_v.at[pl.ds(0, tail_c)], ccntacc_sh.at[t_idx],
                                add=True)

        plsc.subcore_barrier()

        @pl.when(sid == 0)
        def _():
            pltpu.sync_copy(vacc_sh, vsum_hbm.at[cid])
            pltpu.sync_copy(vcntacc_sh, vcnt_hbm.at[cid])
            pltpu.sync_copy(cacc_sh, csum_hbm.at[cid])
            pltpu.sync_copy(ccntacc_sh, ccnt_hbm.at[cid])

    return kern(v_s, c_s, vcls2d, ccls2d, vcls1d, ccls1d)


def _mha_body(sem, sum2, cnt2, Wqkv, bqkv, Wo, bo):
    fea_sum = sum2[0] + sum2[1]
    cnt = cnt2[0][:, 0:1] + cnt2[1][:, 0:1] + 1e-8
    fea = fea_sum / cnt

    dn_t = (((1,), (1,)), ((), ()))
    q = lax.dot_general(sem, Wqkv[0:EMB], dn_t,
                        preferred_element_type=jnp.float32) + bqkv[0, 0:EMB]
    k = lax.dot_general(fea, Wqkv[EMB:2 * EMB], dn_t,
                        preferred_element_type=jnp.float32) + bqkv[0, EMB:2 * EMB]
    v = lax.dot_general(fea, Wqkv[2 * EMB:3 * EMB], dn_t,
                        preferred_element_type=jnp.float32) + bqkv[0, 2 * EMB:3 * EMB]

    outs = []
    scale = 1.0 / (HD ** 0.5)
    for h in range(NHEADS):
        qh = q[:, h * HD:(h + 1) * HD]
        kh = k[:, h * HD:(h + 1) * HD]
        vh = v[:, h * HD:(h + 1) * HD]
        scores = lax.dot_general(qh, kh, dn_t,
                                 preferred_element_type=jnp.float32) * scale
        m = jnp.max(scores, axis=-1, keepdims=True)
        e = jnp.exp(scores - m)
        attn = e / jnp.sum(e, axis=-1, keepdims=True)
        outs.append(lax.dot_general(attn, vh, (((1,), (0,)), ((), ())),
                                    preferred_element_type=jnp.float32))
    o = jnp.concatenate(outs, axis=1)
    return lax.dot_general(o, Wo, dn_t,
                           preferred_element_type=jnp.float32) + bo[0]


def _attn_tc(vsum, vcnt, csum, ccnt, v_sem, c_sem,
             Wqkv_v, bqkv_v, Wo_v, bo_v, Wqkv_c, bqkv_c, Wo_c, bo_c):
    def body(vsum_r, vcnt_r, csum_r, ccnt_r, vsem_r, csem_r,
             wqv_r, bqv_r, wov_r, bov_r, wqc_r, bqc_r, woc_r, boc_r,
             vfin_r, cfin_r):
        vfin_r[...] = _mha_body(vsem_r[...], vsum_r[...], vcnt_r[...],
                                wqv_r[...], bqv_r[...], wov_r[...], bov_r[...])
        cfin_r[...] = _mha_body(csem_r[...], csum_r[...], ccnt_r[...],
                                wqc_r[...], bqc_r[...], woc_r[...], boc_r[...])

    return pl.pallas_call(
        body,
        out_shape=[jax.ShapeDtypeStruct((NCLS, EMB), jnp.float32),
                   jax.ShapeDtypeStruct((NCLS, EMB), jnp.float32)],
    )(vsum, vcnt, csum, ccnt, v_sem, c_sem,
      Wqkv_v, bqkv_v.reshape(1, -1), Wo_v, bo_v.reshape(1, -1),
      Wqkv_c, bqkv_c.reshape(1, -1), Wo_c, bo_c.reshape(1, -1))


def _gather_sc(vfin, cfin, vcls2d, ccls2d, vcls1d, ccls1d, nv, nc):
    nch_v, nx_v, tail_v = _splits(nv)
    nch_c, nx_c, tail_c = _splits(nc)

    out_type = [
        jax.ShapeDtypeStruct((nv, EMB), jnp.float32),
        jax.ShapeDtypeStruct((nc, EMB), jnp.float32),
    ]
    scratch_types = [
        pltpu.VMEM_SHARED((NCLS, EMB), jnp.float32),
        pltpu.VMEM_SHARED((NCLS, EMB), jnp.float32),
        pltpu.VMEM((CH, EMB), jnp.float32),
        pltpu.VMEM((CH, EMB), jnp.float32),
        pltpu.VMEM((KSUB, BLK), jnp.int32),
        pltpu.VMEM((KSUB, BLK), jnp.int32),
        pltpu.SemaphoreType.DMA,
        pltpu.SemaphoreType.DMA,
        pltpu.SemaphoreType.DMA,
        pltpu.SemaphoreType.DMA,
        pltpu.SemaphoreType.DMA,
        pltpu.SemaphoreType.DMA,
    ]
    if nx_v:
        scratch_types += [pltpu.VMEM((KSUB, BLK), jnp.int32),
                          pltpu.VMEM((BLK, EMB), jnp.float32)]
    if tail_v:
        scratch_types += [pltpu.VMEM((tail_v,), jnp.int32),
                          pltpu.VMEM((tail_v, EMB), jnp.float32)]
    if tail_c:
        scratch_types += [pltpu.VMEM((tail_c,), jnp.int32),
                          pltpu.VMEM((tail_c, EMB), jnp.float32)]

    @functools.partial(pl.kernel, mesh=_mesh(), out_type=out_type,
                       scratch_types=scratch_types)
    def kern(vfin_hbm, cfin_hbm, vcls2_hbm, ccls2_hbm, vcls1_hbm, ccls1_hbm,
             vout_hbm, cout_hbm,
             vfin_sh, cfin_sh, rows0, rows1, idx0, idx1,
             s_i0, s_i1, s_g0, s_g1, s_w0, s_w1, *extra):
        cid = lax.axis_index("core")
        sid = lax.axis_index("subcore")
        wid = sid * NCORES + cid

        @pl.when(sid == 0)
        def _():
            pltpu.sync_copy(vfin_hbm, vfin_sh)
            pltpu.sync_copy(cfin_hbm, cfin_sh)

        plsc.subcore_barrier()

        def branch(fin_sh, cls2_hbm, out_hbm, nch):
            def work(ch, idxb, rowsb, s_i, s_g, s_w):
                base = pl.multiple_of(ch * CH, CH)
                pltpu.make_async_copy(cls2_hbm.at[ch], idxb, s_i).wait()

                @pl.when(ch >= wid + 2 * NW)
                def _():
                    pltpu.make_async_copy(
                        rowsb, out_hbm.at[pl.ds(base, CH)], s_w).wait()

                hs = [pltpu.async_copy(fin_sh.at[idxb.at[j]],
                                       rowsb.at[pl.ds(j * BLK, BLK)], s_g)
                      for j in range(KSUB)]
                for h in hs:
                    h.wait()
                pltpu.async_copy(rowsb, out_hbm.at[pl.ds(base, CH)], s_w)

                @pl.when(ch + 2 * NW < nch)
                def _():
                    pltpu.async_copy(cls2_hbm.at[ch + 2 * NW], idxb, s_i)

            @pl.when(wid < nch)
            def _():
                pltpu.async_copy(cls2_hbm.at[wid], idx0, s_i0)

            @pl.when(wid + NW < nch)
            def _():
                pltpu.async_copy(cls2_hbm.at[wid + NW], idx1, s_i1)

            @pl.loop(wid, nch, step=2 * NW)
            def _(ch):
                work(ch, idx0, rows0, s_i0, s_g0, s_w0)

                @pl.when(ch + NW < nch)
                def _():
                    work(ch + NW, idx1, rows1, s_i1, s_g1, s_w1)

            @pl.when(wid < nch)
            def _():
                pltpu.make_async_copy(rows0, out_hbm.at[pl.ds(0, CH)], s_w0).wait()

            @pl.when(wid + NW < nch)
            def _():
                pltpu.make_async_copy(rows1, out_hbm.at[pl.ds(0, CH)], s_w1).wait()

        branch(vfin_sh, vcls2_hbm, vout_hbm, nch_v)
        branch(cfin_sh, ccls2_hbm, cout_hbm, nch_c)

        ei = 0
        for x in range(nx_v):
            e_idx, e_rows = extra[ei], extra[ei + 1]
            ei += 2
            base = nch_v * CH + x * BLK

            @pl.when((cid == 0) & (sid == 2 + x))
            def _():
                pltpu.sync_copy(vcls2_hbm.at[nch_v], e_idx)
                pltpu.sync_copy(vfin_sh.at[e_idx.at[x]], e_rows)
                pltpu.sync_copy(e_rows, vout_hbm.at[pl.ds(base, BLK)])

        if tail_v:
            t_idx, t_rows = extra[ei], extra[ei + 1]
            ei += 2
            base = nch_v * CH + nx_v * BLK

            @pl.when((cid == 0) & (sid == 1))
            def _():
                pltpu.sync_copy(vcls1_hbm.at[0, pl.ds(base, tail_v)], t_idx)
                pltpu.sync_copy(vfin_sh.at[t_idx], t_rows)
                pltpu.sync_copy(t_rows, vout_hbm.at[pl.ds(base, tail_v)])
        if tail_c:
            t_idx, t_rows = extra[ei], extra[ei + 1]
            base = nch_c * CH + nx_c * BLK

            @pl.when((cid == 1) & (sid == 1))
            def _():
                pltpu.sync_copy(ccls1_hbm.at[0, pl.ds(base, tail_c)], t_idx)
                pltpu.sync_copy(cfin_sh.at[t_idx], t_rows)
                pltpu.sync_copy(t_rows, cout_hbm.at[pl.ds(base, tail_c)])

    return kern(vfin, cfin, vcls2d, ccls2d, vcls1d, ccls1d)


def kernel(v_s, c_s, v_sem, c_sem, v_class, c_class,
           Wqkv_v, bqkv_v, Wo_v, bo_v, Wqkv_c, bqkv_c, Wo_c, bo_c):
    nv = v_s.shape[0]
    nc = c_s.shape[0]
    pad_v = (-nv) % CH
    pad_c = (-nc) % CH
    vcls2d = jnp.pad(v_class, (0, pad_v)).reshape(-1, KSUB, BLK)
    ccls2d = jnp.pad(c_class, (0, pad_c)).reshape(-1, KSUB, BLK)
    vcls1d = v_class.reshape(1, nv)
    ccls1d = c_class.reshape(1, nc)

    vsum, vcnt, csum, ccnt = _segment_sums_sc(v_s, c_s, vcls2d, ccls2d,
                                              vcls1d, ccls1d)
    vfin, cfin = _attn_tc(vsum, vcnt, csum, ccnt, v_sem, c_sem,
                          Wqkv_v, bqkv_v, Wo_v, bo_v,
                          Wqkv_c, bqkv_c, Wo_c, bo_c)
    v_updates, c_updates = _gather_sc(vfin, cfin, vcls2d, ccls2d,
                                      vcls1d, ccls1d, nv, nc)
    return (v_updates, c_updates)


# speedup vs baseline: 7.4426x; 1.2353x over previous
import dataclasses
import functools

import jax
import jax.numpy as jnp
from jax import lax
from jax.experimental import pallas as pl
from jax.experimental.pallas import tpu as pltpu
from jax.experimental.pallas import tpu_sc as plsc

EMB = 128
NCLS = 16
NHEADS = 4
HD = EMB // NHEADS
BLK = 128
KSUB = 2
CH = KSUB * BLK
CNTW = 128

NCORES = 2
NSUB = 16
NW = NCORES * NSUB


@functools.cache
def _mesh():
    return plsc.VectorSubcoreMesh(core_axis_name="core", subcore_axis_name="subcore")


@functools.cache
def _sc_params():
    cp = pltpu.CompilerParams()
    if "needs_layout_passes" in pltpu.CompilerParams.__dataclass_fields__:
        cp = dataclasses.replace(cp, needs_layout_passes=False)
    return cp


def _splits(n):
    nch = n // CH
    rem = n - nch * CH
    nx = rem // BLK
    tail = rem - nx * BLK
    assert tail % 8 == 0
    return nch, nx, tail


def _fill2d(ref, rows, cols, value):
    val = jnp.full((16,), value, jnp.float32)

    @pl.loop(0, rows)
    def _(i):
        for j in range(cols // 16):
            ref[i, pl.ds(j * 16, 16)] = val


def _segment_sums_sc(v_s, c_s, vcls2d, ccls2d, vcls1d, ccls1d):
    nv = v_s.shape[0]
    nc = c_s.shape[0]
    nch_v, nx_v, tail_v = _splits(nv)
    nch_c, nx_c, tail_c = _splits(nc)

    out_type = [
        jax.ShapeDtypeStruct((NCORES, NCLS, EMB), jnp.float32),
        jax.ShapeDtypeStruct((NCORES, NSUB, NCLS), jnp.float32),
        jax.ShapeDtypeStruct((NCORES, NCLS, EMB), jnp.float32),
        jax.ShapeDtypeStruct((NCORES, NSUB, NCLS), jnp.float32),
    ]
    scratch_types = [
        pltpu.VMEM_SHARED((NCLS, EMB), jnp.float32),
        pltpu.VMEM_SHARED((NCLS, EMB), jnp.float32),
        pltpu.VMEM((CH, EMB), jnp.float32),
        pltpu.VMEM((CH, EMB), jnp.float32),
        pltpu.VMEM((KSUB, BLK), jnp.int32),
        pltpu.VMEM((KSUB, BLK), jnp.int32),
        pltpu.VMEM((NCLS,), jnp.float32),
        pltpu.VMEM((NCLS,), jnp.float32),
        pltpu.VMEM((NCLS, EMB), jnp.float32),
        pltpu.SemaphoreType.DMA,
        pltpu.SemaphoreType.DMA,
        pltpu.SemaphoreType.DMA,
        pltpu.SemaphoreType.DMA,
        pltpu.SemaphoreType.DMA,
        pltpu.SemaphoreType.DMA,
    ]
    if nx_v:
        scratch_types += [pltpu.VMEM((KSUB, BLK), jnp.int32),
                          pltpu.VMEM((BLK, EMB), jnp.float32)]
    if tail_v:
        scratch_types += [pltpu.VMEM((tail_v,), jnp.int32),
                          pltpu.VMEM((tail_v, EMB), jnp.float32)]
    if tail_c:
        scratch_types += [pltpu.VMEM((tail_c,), jnp.int32),
                          pltpu.VMEM((tail_c, EMB), jnp.float32)]

    @functools.partial(pl.kernel, mesh=_mesh(), out_type=out_type,
                       scratch_types=scratch_types,
                       compiler_params=_sc_params())
    def kern(v_s_hbm, c_s_hbm, vcls2_hbm, ccls2_hbm, vcls1_hbm, ccls1_hbm,
             vsum_hbm, vcnt_hbm, csum_hbm, ccnt_hbm,
             vacc_sh, cacc_sh,
             rows0, rows1, idx0, idx1, vcnt_loc, ccnt_loc, zero_w,
             s_i0, s_r0, s_i1, s_r1, s_s0, s_s1, *extra):
        cid = lax.axis_index("core")
        sid = lax.axis_index("subcore")
        wid = sid * NCORES + cid

        ones16 = jnp.ones((NCLS,), jnp.float32)
        vcnt_loc[...] = jnp.zeros((NCLS,), jnp.float32)
        ccnt_loc[...] = jnp.zeros((NCLS,), jnp.float32)

        def count_ids(vec16, cnt_ref):
            plsc.addupdate_scatter(cnt_ref, [vec16], ones16)

        @pl.when(sid == 0)
        def _():
            _fill2d(zero_w, NCLS, EMB, 0.0)
            pltpu.sync_copy(zero_w, vacc_sh)
            pltpu.sync_copy(zero_w, cacc_sh)

        plsc.subcore_barrier()

        def branch(rows_hbm, cls2_hbm, acc_sh, cnt_ref, nch):
            def start(ch, idxb, rowsb, s_i, s_r):
                base = pl.multiple_of(ch * CH, CH)
                pltpu.async_copy(cls2_hbm.at[ch], idxb, s_i)
                pltpu.async_copy(rows_hbm.at[pl.ds(base, CH)], rowsb, s_r)

            def wait(ch, idxb, rowsb, s_i, s_r):
                base = pl.multiple_of(ch * CH, CH)
                pltpu.make_async_copy(cls2_hbm.at[ch], idxb, s_i).wait()
                pltpu.make_async_copy(
                    rows_hbm.at[pl.ds(base, CH)], rowsb, s_r).wait()

            def work(ch, idxb, rowsb, s_i, s_r, s_s):
                wait(ch, idxb, rowsb, s_i, s_r)
                hs = []
                for j in range(KSUB):
                    hs.append(pltpu.async_copy(
                        rowsb.at[pl.ds(j * BLK, BLK)],
                        acc_sh.at[idxb.at[j]], s_s, add=True))
                for j in range(KSUB):
                    for g in range(BLK // NCLS):
                        count_ids(idxb[j, pl.ds(g * NCLS, NCLS)], cnt_ref)
                for h in hs:
                    h.wait()

                @pl.when(ch + 2 * NW < nch)
                def _():
                    start(ch + 2 * NW, idxb, rowsb, s_i, s_r)

            @pl.when(wid < nch)
            def _():
                start(wid, idx0, rows0, s_i0, s_r0)

            @pl.when(wid + NW < nch)
            def _():
                start(wid + NW, idx1, rows1, s_i1, s_r1)

            @pl.loop(wid, nch, step=2 * NW)
            def _(ch):
                work(ch, idx0, rows0, s_i0, s_r0, s_s0)

                @pl.when(ch + NW < nch)
                def _():
                    work(ch + NW, idx1, rows1, s_i1, s_r1, s_s1)

        branch(v_s_hbm, vcls2_hbm, vacc_sh, vcnt_loc, nch_v)
        branch(c_s_hbm, ccls2_hbm, cacc_sh, ccnt_loc, nch_c)

        ei = 0
        for x in range(nx_v):
            e_idx, e_rows = extra[ei], extra[ei + 1]
            ei += 2
            base = nch_v * CH + x * BLK

            @pl.when((cid == 0) & (sid == 2 + x))
            def _():
                pltpu.sync_copy(vcls2_hbm.at[nch_v], e_idx)
                pltpu.sync_copy(v_s_hbm.at[pl.ds(base, BLK)], e_rows)
                pltpu.sync_copy(e_rows, vacc_sh.at[e_idx.at[x]], add=True)
                for g in range(BLK // NCLS):
                    count_ids(e_idx[x, pl.ds(g * NCLS, NCLS)], vcnt_loc)

        if tail_v:
            t_idx, t_rows = extra[ei], extra[ei + 1]
            ei += 2
            base = nch_v * CH + nx_v * BLK

            @pl.when((cid == 0) & (sid == 1))
            def _():
                pltpu.sync_copy(vcls1_hbm.at[0, pl.ds(base, tail_v)], t_idx)
                pltpu.sync_copy(v_s_hbm.at[pl.ds(base, tail_v)], t_rows)
                pltpu.sync_copy(t_rows, vacc_sh.at[t_idx], add=True)
                for g in range(tail_v // NCLS):
                    count_ids(t_idx[pl.ds(g * NCLS, NCLS)], vcnt_loc)
        if tail_c:
            t_idx, t_rows = extra[ei], extra[ei + 1]
            base = nch_c * CH + nx_c * BLK

            @pl.when((cid == 1) & (sid == 1))
            def _():
                pltpu.sync_copy(ccls1_hbm.at[0, pl.ds(base, tail_c)], t_idx)
                pltpu.sync_copy(c_s_hbm.at[pl.ds(base, tail_c)], t_rows)
                pltpu.sync_copy(t_rows, cacc_sh.at[t_idx], add=True)
                for g in range(tail_c // NCLS):
                    count_ids(t_idx[pl.ds(g * NCLS, NCLS)], ccnt_loc)

        pltpu.sync_copy(vcnt_loc, vcnt_hbm.at[cid, sid])
        pltpu.sync_copy(ccnt_loc, ccnt_hbm.at[cid, sid])

        plsc.subcore_barrier()

        @pl.when(sid == 0)
        def _():
            pltpu.sync_copy(vacc_sh, vsum_hbm.at[cid])
            pltpu.sync_copy(cacc_sh, csum_hbm.at[cid])

    return kern(v_s, c_s, vcls2d, ccls2d, vcls1d, ccls1d)


def _mha_body(sem, sum2, cnt2, Wqkv, bqkv, Wo, bo):
    fea_sum = sum2[0] + sum2[1]
    cnt = jnp.sum(cnt2.reshape(NCORES * NSUB, NCLS), axis=0)
    recip = 1.0 / (cnt[None, :] + 1e-8)
    rr = lax.broadcasted_iota(jnp.int32, (NCLS, NCLS), 0)
    cc = lax.broadcasted_iota(jnp.int32, (NCLS, NCLS), 1)
    diag_inv = jnp.where(rr == cc, jnp.broadcast_to(recip, (NCLS, NCLS)), 0.0)
    fea = lax.dot_general(diag_inv, fea_sum, (((1,), (0,)), ((), ())),
                          preferred_element_type=jnp.float32)

    dn_t = (((1,), (1,)), ((), ()))
    q = lax.dot_general(sem, Wqkv[0:EMB], dn_t,
                        preferred_element_type=jnp.float32) + bqkv[0, 0:EMB]
    k = lax.dot_general(fea, Wqkv[EMB:2 * EMB], dn_t,
                        preferred_element_type=jnp.float32) + bqkv[0, EMB:2 * EMB]
    v = lax.dot_general(fea, Wqkv[2 * EMB:3 * EMB], dn_t,
                        preferred_element_type=jnp.float32) + bqkv[0, 2 * EMB:3 * EMB]

    outs = []
    scale = 1.0 / (HD ** 0.5)
    for h in range(NHEADS):
        qh = q[:, h * HD:(h + 1) * HD]
        kh = k[:, h * HD:(h + 1) * HD]
        vh = v[:, h * HD:(h + 1) * HD]
        scores = lax.dot_general(qh, kh, dn_t,
                                 preferred_element_type=jnp.float32) * scale
        m = jnp.max(scores, axis=-1, keepdims=True)
        e = jnp.exp(scores - m)
        attn = e / jnp.sum(e, axis=-1, keepdims=True)
        outs.append(lax.dot_general(attn, vh, (((1,), (0,)), ((), ())),
                                    preferred_element_type=jnp.float32))
    o = jnp.concatenate(outs, axis=1)
    return lax.dot_general(o, Wo, dn_t,
                           preferred_element_type=jnp.float32) + bo[0]


def _attn_tc(vsum, vcnt, csum, ccnt, v_sem, c_sem,
             Wqkv_v, bqkv_v, Wo_v, bo_v, Wqkv_c, bqkv_c, Wo_c, bo_c):
    def body(vsum_r, vcnt_r, csum_r, ccnt_r, vsem_r, csem_r,
             wqv_r, bqv_r, wov_r, bov_r, wqc_r, bqc_r, woc_r, boc_r,
             vfin_r, cfin_r):
        vfin_r[...] = _mha_body(vsem_r[...], vsum_r[...], vcnt_r[...],
                                wqv_r[...], bqv_r[...], wov_r[...], bov_r[...])
        cfin_r[...] = _mha_body(csem_r[...], csum_r[...], ccnt_r[...],
                                wqc_r[...], bqc_r[...], woc_r[...], boc_r[...])

    return pl.pallas_call(
        body,
        out_shape=[jax.ShapeDtypeStruct((NCLS, EMB), jnp.float32),
                   jax.ShapeDtypeStruct((NCLS, EMB), jnp.float32)],
    )(vsum, vcnt, csum, ccnt, v_sem, c_sem,
      Wqkv_v, bqkv_v.reshape(1, -1), Wo_v, bo_v.reshape(1, -1),
      Wqkv_c, bqkv_c.reshape(1, -1), Wo_c, bo_c.reshape(1, -1))


def _gather_sc(vfin, cfin, vcls2d, ccls2d, vcls1d, ccls1d, nv, nc):
    nch_v, nx_v, tail_v = _splits(nv)
    nch_c, nx_c, tail_c = _splits(nc)

    out_type = [
        jax.ShapeDtypeStruct((nv, EMB), jnp.float32),
        jax.ShapeDtypeStruct((nc, EMB), jnp.float32),
    ]
    scratch_types = [
        pltpu.VMEM_SHARED((NCLS, EMB), jnp.float32),
        pltpu.VMEM_SHARED((NCLS, EMB), jnp.float32),
        pltpu.VMEM((CH, EMB), jnp.float32),
        pltpu.VMEM((CH, EMB), jnp.float32),
        pltpu.VMEM((KSUB, BLK), jnp.int32),
        pltpu.VMEM((KSUB, BLK), jnp.int32),
        pltpu.SemaphoreType.DMA,
        pltpu.SemaphoreType.DMA,
        pltpu.SemaphoreType.DMA,
        pltpu.SemaphoreType.DMA,
        pltpu.SemaphoreType.DMA,
        pltpu.SemaphoreType.DMA,
    ]
    if nx_v:
        scratch_types += [pltpu.VMEM((KSUB, BLK), jnp.int32),
                          pltpu.VMEM((BLK, EMB), jnp.float32)]
    if tail_v:
        scratch_types += [pltpu.VMEM((tail_v,), jnp.int32),
                          pltpu.VMEM((tail_v, EMB), jnp.float32)]
    if tail_c:
        scratch_types += [pltpu.VMEM((tail_c,), jnp.int32),
                          pltpu.VMEM((tail_c, EMB), jnp.float32)]

    @functools.partial(pl.kernel, mesh=_mesh(), out_type=out_type,
                       scratch_types=scratch_types)
    def kern(vfin_hbm, cfin_hbm, vcls2_hbm, ccls2_hbm, vcls1_hbm, ccls1_hbm,
             vout_hbm, cout_hbm,
             vfin_sh, cfin_sh, rows0, rows1, idx0, idx1,
             s_i0, s_i1, s_g0, s_g1, s_w0, s_w1, *extra):
        cid = lax.axis_index("core")
        sid = lax.axis_index("subcore")
        wid = sid * NCORES + cid

        @pl.when(sid == 0)
        def _():
            pltpu.sync_copy(vfin_hbm, vfin_sh)
            pltpu.sync_copy(cfin_hbm, cfin_sh)

        plsc.subcore_barrier()

        def branch(fin_sh, cls2_hbm, out_hbm, nch):
            def work(ch, idxb, rowsb, s_i, s_g, s_w):
                base = pl.multiple_of(ch * CH, CH)
                pltpu.make_async_copy(cls2_hbm.at[ch], idxb, s_i).wait()

                @pl.when(ch >= wid + 2 * NW)
                def _():
                    pltpu.make_async_copy(
                        rowsb, out_hbm.at[pl.ds(base, CH)], s_w).wait()

                hs = [pltpu.async_copy(fin_sh.at[idxb.at[j]],
                                       rowsb.at[pl.ds(j * BLK, BLK)], s_g)
                      for j in range(KSUB)]
                for h in hs:
                    h.wait()
                pltpu.async_copy(rowsb, out_hbm.at[pl.ds(base, CH)], s_w)

                @pl.when(ch + 2 * NW < nch)
                def _():
                    pltpu.async_copy(cls2_hbm.at[ch + 2 * NW], idxb, s_i)

            @pl.when(wid < nch)
            def _():
                pltpu.async_copy(cls2_hbm.at[wid], idx0, s_i0)

            @pl.when(wid + NW < nch)
            def _():
                pltpu.async_copy(cls2_hbm.at[wid + NW], idx1, s_i1)

            @pl.loop(wid, nch, step=2 * NW)
            def _(ch):
                work(ch, idx0, rows0, s_i0, s_g0, s_w0)

                @pl.when(ch + NW < nch)
                def _():
                    work(ch + NW, idx1, rows1, s_i1, s_g1, s_w1)

            @pl.when(wid < nch)
            def _():
                pltpu.make_async_copy(rows0, out_hbm.at[pl.ds(0, CH)], s_w0).wait()

            @pl.when(wid + NW < nch)
            def _():
                pltpu.make_async_copy(rows1, out_hbm.at[pl.ds(0, CH)], s_w1).wait()

        branch(vfin_sh, vcls2_hbm, vout_hbm, nch_v)
        branch(cfin_sh, ccls2_hbm, cout_hbm, nch_c)

        ei = 0
        for x in range(nx_v):
            e_idx, e_rows = extra[ei], extra[ei + 1]
            ei += 2
            base = nch_v * CH + x * BLK

            @pl.when((cid == 0) & (sid == 2 + x))
            def _():
                pltpu.sync_copy(vcls2_hbm.at[nch_v], e_idx)
                pltpu.sync_copy(vfin_sh.at[e_idx.at[x]], e_rows)
                pltpu.sync_copy(e_rows, vout_hbm.at[pl.ds(base, BLK)])

        if tail_v:
            t_idx, t_rows = extra[ei], extra[ei + 1]
            ei += 2
            base = nch_v * CH + nx_v * BLK

            @pl.when((cid == 0) & (sid == 1))
            def _():
                pltpu.sync_copy(vcls1_hbm.at[0, pl.ds(base, tail_v)], t_idx)
                pltpu.sync_copy(vfin_sh.at[t_idx], t_rows)
                pltpu.sync_copy(t_rows, vout_hbm.at[pl.ds(base, tail_v)])
        if tail_c:
            t_idx, t_rows = extra[ei], extra[ei + 1]
            base = nch_c * CH + nx_c * BLK

            @pl.when((cid == 1) & (sid == 1))
            def _():
                pltpu.sync_copy(ccls1_hbm.at[0, pl.ds(base, tail_c)], t_idx)
                pltpu.sync_copy(cfin_sh.at[t_idx], t_rows)
                pltpu.sync_copy(t_rows, cout_hbm.at[pl.ds(base, tail_c)])

    return kern(vfin, cfin, vcls2d, ccls2d, vcls1d, ccls1d)


def kernel(v_s, c_s, v_sem, c_sem, v_class, c_class,
           Wqkv_v, bqkv_v, Wo_v, bo_v, Wqkv_c, bqkv_c, Wo_c, bo_c):
    nv = v_s.shape[0]
    nc = c_s.shape[0]
    pad_v = (-nv) % CH
    pad_c = (-nc) % CH
    vcls2d = jnp.pad(v_class, (0, pad_v)).reshape(-1, KSUB, BLK)
    ccls2d = jnp.pad(c_class, (0, pad_c)).reshape(-1, KSUB, BLK)
    vcls1d = v_class.reshape(1, nv)
    ccls1d = c_class.reshape(1, nc)

    vsum, vcnt, csum, ccnt = _segment_sums_sc(v_s, c_s, vcls2d, ccls2d,
                                              vcls1d, ccls1d)
    vfin, cfin = _attn_tc(vsum, vcnt, csum, ccnt, v_sem, c_sem,
                          Wqkv_v, bqkv_v, Wo_v, bo_v,
                          Wqkv_c, bqkv_c, Wo_c, bo_c)
    v_updates, c_updates = _gather_sc(vfin, cfin, vcls2d, ccls2d,
                                      vcls1d, ccls1d, nv, nc)
    return (v_updates, c_updates)
